# baseline (speedup 1.0000x reference)
import jax
import jax.numpy as jnp
from jax.experimental import pallas as pl
from jax.experimental.pallas import tpu as pltpu

_TV = 800


def _temp_body(wl_ref, xt_ref, bl_ref, rt_ref):
    tl = jnp.dot(wl_ref[...], xt_ref[...],
                 preferred_element_type=jnp.float32) + bl_ref[...]
    rt = 1.0 / jax.nn.sigmoid(tl)
    rt_ref[...] = jnp.broadcast_to(rt, rt_ref.shape)


def _main_body(w0_ref, w1_ref, w2_ref, w3_ref, xtb_ref, rt_ref, b_ref, o_ref):
    h = w0_ref.shape[0]
    for i, w_ref in enumerate((w0_ref, w1_ref, w2_ref, w3_ref)):
        acc = jnp.dot(w_ref[...], xtb_ref[...],
                      preferred_element_type=jnp.float32)
        o_ref[i * h:(i + 1) * h, :] = (
            acc + b_ref[i * h:(i + 1) * h, :]) * rt_ref[0:1, :]


def kernel(inputs, W, b):
    x = inputs
    n, k = x.shape
    v = W.shape[1] - 1

    wt = W.T
    xt = x.T
    xtb = xt.astype(jnp.bfloat16)
    wl = wt[v:v + 1, :]
    bl = b[-1].reshape(1, 1)
    b2 = b[:-1].reshape(v, 1)

    rt = pl.pallas_call(
        _temp_body,
        out_shape=jax.ShapeDtypeStruct((8, n), jnp.float32),
    )(wl, xt, bl)

    out_t = pl.pallas_call(
        _main_body,
        grid=(pl.cdiv(v, _TV),),
        in_specs=[
            pl.BlockSpec((_TV // 4, k), lambda j: (4 * j, 0)),
            pl.BlockSpec((_TV // 4, k), lambda j: (4 * j + 1, 0)),
            pl.BlockSpec((_TV // 4, k), lambda j: (4 * j + 2, 0)),
            pl.BlockSpec((_TV // 4, k), lambda j: (4 * j + 3, 0)),
            pl.BlockSpec((k, n), lambda j: (0, 0)),
            pl.BlockSpec((8, n), lambda j: (0, 0)),
            pl.BlockSpec((_TV, 1), lambda j: (j, 0)),
        ],
        out_specs=pl.BlockSpec((_TV, n), lambda j: (j, 0)),
        out_shape=jax.ShapeDtypeStruct((v, n), jnp.float32),
        compiler_params=pltpu.CompilerParams(
            dimension_semantics=("parallel",)),
    )(wt, wt, wt, wt, xtb, rt, b2)
    return out_t.T


# speedup vs baseline: 1.0551x; 1.0551x over previous
import jax
import jax.numpy as jnp
from jax.experimental import pallas as pl
from jax.experimental.pallas import tpu as pltpu

_TV = 1024


def _pre_body(x_ref, wl_ref, bl_ref, rt_ref, xtb_ref):
    tl = jax.lax.dot_general(
        wl_ref[...], x_ref[...], (((1,), (1,)), ((), ())),
        preferred_element_type=jnp.float32) + bl_ref[...]
    rt = 1.0 / jax.nn.sigmoid(tl)
    rt_ref[...] = jnp.broadcast_to(rt, rt_ref.shape)
    xtb_ref[...] = x_ref[...].astype(jnp.bfloat16).T


def _main_body(wt_ref, xtb_ref, rt_ref, b_ref, o_ref):
    acc = jnp.dot(wt_ref[...], xtb_ref[...],
                  preferred_element_type=jnp.float32)
    o_ref[...] = (acc + b_ref[...]) * rt_ref[0:1, :]


def kernel(inputs, W, b):
    x = inputs
    n, k = x.shape
    v = W.shape[1] - 1

    wt = W.T
    wl = wt[v:v + 1, :]
    bl = b[-1].reshape(1, 1)
    b2 = b[:-1].reshape(v, 1)

    rt, xtb = pl.pallas_call(
        _pre_body,
        out_shape=(jax.ShapeDtypeStruct((8, n), jnp.float32),
                   jax.ShapeDtypeStruct((k, n), jnp.bfloat16)),
    )(x, wl, bl)

    out_t = pl.pallas_call(
        _main_body,
        grid=(pl.cdiv(v, _TV),),
        in_specs=[
            pl.BlockSpec((_TV, k), lambda j: (j, 0)),
            pl.BlockSpec((k, n), lambda j: (0, 0)),
            pl.BlockSpec((8, n), lambda j: (0, 0)),
            pl.BlockSpec((_TV, 1), lambda j: (j, 0)),
        ],
        out_specs=pl.BlockSpec((_TV, n), lambda j: (j, 0)),
        out_shape=jax.ShapeDtypeStruct((v, n), jnp.float32),
        compiler_params=pltpu.CompilerParams(
            dimension_semantics=("parallel",)),
    )(wt, xtb, rt, b2)
    return out_t.T
